# baseline (speedup 1.0000x reference)
import functools

import jax
import jax.numpy as jnp
from jax.experimental import pallas as pl
from jax.experimental.pallas import tpu as pltpu


def _gcn_body(a0_ref, a1_ref, x_ref, w_ref, b_ref, out_ref, support_ref):
    @pl.when(pl.program_id(0) == 0)
    def _():
        support_ref[...] = jnp.dot(
            x_ref[...], w_ref[...], preferred_element_type=jnp.float32
        )

    bm = a0_ref.shape[0]
    s = support_ref[...]
    b = b_ref[...]
    out_ref[:bm, :] = (
        jnp.dot(a0_ref[...], s, preferred_element_type=jnp.float32) + b
    )
    out_ref[bm:, :] = (
        jnp.dot(a1_ref[...], s, preferred_element_type=jnp.float32) + b
    )


@functools.partial(jax.jit, static_argnames=("block_m",))
def _gcn(adjacency, input_feature, weight, bias2d, block_m=200):
    n, _ = adjacency.shape
    d_in, d_out = weight.shape
    grid = (n // (2 * block_m),)
    out = pl.pallas_call(
        _gcn_body,
        grid=grid,
        in_specs=[
            pl.BlockSpec((block_m, n), lambda i: (2 * i, 0)),
            pl.BlockSpec((block_m, n), lambda i: (2 * i + 1, 0)),
            pl.BlockSpec((n, d_in), lambda i: (0, 0)),
            pl.BlockSpec((d_in, d_out), lambda i: (0, 0)),
            pl.BlockSpec((1, d_out), lambda i: (0, 0)),
        ],
        out_specs=pl.BlockSpec((2 * block_m, d_out), lambda i: (i, 0)),
        out_shape=jax.ShapeDtypeStruct((n, d_out), jnp.float32),
        scratch_shapes=[pltpu.VMEM((n, d_out), jnp.float32)],
    )(adjacency, adjacency, input_feature, weight, bias2d)
    return out


def kernel(adjacency, input_feature, weight, bias):
    out = _gcn(adjacency, input_feature, weight, bias.reshape(1, -1))
    return (out, weight, bias, adjacency)


# speedup vs baseline: 1.0002x; 1.0002x over previous
import functools

import jax
import jax.numpy as jnp
from jax.experimental import pallas as pl
from jax.experimental.pallas import tpu as pltpu


def _gcn_body(a_ref, x_ref, w_ref, b_ref, out_ref, support_ref):
    @pl.when(pl.program_id(0) == 0)
    def _():
        support_ref[...] = jnp.dot(
            x_ref[...], w_ref[...], preferred_element_type=jnp.float32
        )

    out_ref[...] = (
        jnp.dot(a_ref[...], support_ref[...], preferred_element_type=jnp.float32)
        + b_ref[...]
    )


@functools.partial(jax.jit, static_argnames=("block_m",))
def _gcn(adjacency, input_feature, weight, bias2d, block_m=400):
    n, _ = adjacency.shape
    d_in, d_out = weight.shape
    grid = (pl.cdiv(n, block_m),)
    out = pl.pallas_call(
        _gcn_body,
        grid=grid,
        in_specs=[
            pl.BlockSpec((block_m, n), lambda i: (i, 0)),
            pl.BlockSpec((n, d_in), lambda i: (0, 0)),
            pl.BlockSpec((d_in, d_out), lambda i: (0, 0)),
            pl.BlockSpec((1, d_out), lambda i: (0, 0)),
        ],
        out_specs=pl.BlockSpec((block_m, d_out), lambda i: (i, 0)),
        out_shape=jax.ShapeDtypeStruct((n, d_out), jnp.float32),
        scratch_shapes=[pltpu.VMEM((n, d_out), jnp.float32)],
    )(adjacency, input_feature, weight, bias2d)
    return out


def kernel(adjacency, input_feature, weight, bias):
    out = _gcn(adjacency, input_feature, weight, bias.reshape(1, -1))
    return (out, weight, bias, adjacency)


# speedup vs baseline: 1.0017x; 1.0015x over previous
import functools

import jax
import jax.numpy as jnp
from jax.experimental import pallas as pl
from jax.experimental.pallas import tpu as pltpu


def _gcn_body(a_ref, x_ref, w_ref, b_ref, out_ref, support_ref):
    @pl.when(pl.program_id(0) == 0)
    def _():
        support_ref[...] = jnp.dot(
            x_ref[...], w_ref[...], preferred_element_type=jnp.float32
        )

    out_ref[...] = (
        jnp.dot(a_ref[...], support_ref[...], preferred_element_type=jnp.float32)
        + b_ref[...]
    )


@functools.partial(jax.jit, static_argnames=("block_m",))
def _gcn(adjacency, input_feature, weight, bias2d, block_m=200):
    n, _ = adjacency.shape
    d_in, d_out = weight.shape
    grid = (pl.cdiv(n, block_m),)
    out = pl.pallas_call(
        _gcn_body,
        grid=grid,
        in_specs=[
            pl.BlockSpec((block_m, n), lambda i: (i, 0)),
            pl.BlockSpec((n, d_in), lambda i: (0, 0)),
            pl.BlockSpec((d_in, d_out), lambda i: (0, 0)),
            pl.BlockSpec((1, d_out), lambda i: (0, 0)),
        ],
        out_specs=pl.BlockSpec((block_m, d_out), lambda i: (i, 0)),
        out_shape=jax.ShapeDtypeStruct((n, d_out), jnp.float32),
        scratch_shapes=[pltpu.VMEM((n, d_out), jnp.float32)],
    )(adjacency, input_feature, weight, bias2d)
    return out


def kernel(adjacency, input_feature, weight, bias):
    out = _gcn(adjacency, input_feature, weight, bias.reshape(1, -1))
    return (out, weight, bias, adjacency)
